# baseline (speedup 1.0000x reference)
import functools

import jax
import jax.numpy as jnp
from jax import lax
from jax.experimental import pallas as pl
from jax.experimental.pallas import tpu as pltpu
from jax.experimental.pallas import tpu_sc as plsc

E = 320000
D_EMB = 128
D_FEAT = 16
D_OUT = D_EMB + D_FEAT
V = 400
BLK = 80
NBLK = E // BLK
NW = 32
NJ = NBLK // NW
L = 16


def _sc_embed_concat(idx, table, feat):
    mesh = plsc.VectorSubcoreMesh(core_axis_name="core", subcore_axis_name="subcore")

    @functools.partial(
        pl.kernel,
        out_type=jax.ShapeDtypeStruct((E, D_OUT), jnp.float32),
        mesh=mesh,
        scratch_types=[
            pltpu.VMEM((V, D_EMB), jnp.float32),
            pltpu.VMEM((2, BLK), jnp.int32),
            pltpu.VMEM((2, BLK, D_OUT), jnp.float32),
            pltpu.VMEM((2, BLK, D_FEAT), jnp.float32),
        ]
        + [pltpu.SemaphoreType.DMA] * 6,
    )
    def run(i_hbm, t_hbm, f_hbm, o_hbm, t_v, i_v, o_v, f_v,
            is0, is1, fs0, fs1, ws0, ws1):
        wid = lax.axis_index("subcore") * 2 + lax.axis_index("core")
        isems, fsems, wsems = (is0, is1), (fs0, fs1), (ws0, ws1)

        pltpu.sync_copy(t_hbm, t_v)

        def blk_of(j):
            return wid + NW * j

        def start_loads(j, p):
            b = blk_of(j)
            pltpu.async_copy(i_hbm.at[pl.ds(b * BLK, BLK)], i_v.at[p], isems[p])
            pltpu.async_copy(
                f_hbm.at[pl.ds(b * BLK, BLK), :], f_v.at[p], fsems[p]
            )

        def wait_loads(j, p):
            b = blk_of(j)
            pltpu.make_async_copy(
                i_hbm.at[pl.ds(b * BLK, BLK)], i_v.at[p], isems[p]
            ).wait()
            pltpu.make_async_copy(
                f_hbm.at[pl.ds(b * BLK, BLK), :], f_v.at[p], fsems[p]
            ).wait()

        def assemble(p):
            @pl.loop(0, BLK // L)
            def _(g):
                iv = i_v[p, pl.ds(g * L, L)]

                def stores(st):
                    vals, fval, r = st
                    for k in range(D_EMB // L):
                        o_v[p, r, pl.ds(k * L, L)] = vals[k]
                    o_v[p, r, pl.ds(D_EMB, D_FEAT)] = fval

                prev = None
                for e in range(L):
                    r = g * L + e
                    row = iv[e]
                    vals = [t_v[row, pl.ds(k * L, L)] for k in range(D_EMB // L)]
                    fval = f_v[p, r, :]
                    if prev is not None:
                        stores(prev)
                    prev = (vals, fval, r)
                stores(prev)

        start_loads(0, 0)
        start_loads(1, 1)

        @pl.loop(0, NJ)
        def _(j):
            for p in range(2):

                @pl.when(j % 2 == p)
                def _():
                    @pl.when(j >= 2)
                    def _():
                        pltpu.make_async_copy(
                            o_v.at[p], o_hbm.at[pl.ds(0, BLK), :], wsems[p]
                        ).wait()

                    wait_loads(j, p)
                    assemble(p)
                    pltpu.async_copy(
                        o_v.at[p],
                        o_hbm.at[pl.ds(blk_of(j) * BLK, BLK), :],
                        wsems[p],
                    )

                    @pl.when(j + 2 < NJ)
                    def _():
                        start_loads(j + 2, p)

        for p in range(2):
            pltpu.make_async_copy(
                o_v.at[p], o_hbm.at[pl.ds(0, BLK), :], wsems[p]
            ).wait()

        @pl.when(wid < NBLK - NW * NJ)
        def _():
            b = NW * NJ + wid
            pltpu.sync_copy(i_hbm.at[pl.ds(b * BLK, BLK)], i_v.at[0])
            pltpu.sync_copy(f_hbm.at[pl.ds(b * BLK, BLK), :], f_v.at[0])
            assemble(0)
            pltpu.sync_copy(o_v.at[0], o_hbm.at[pl.ds(b * BLK, BLK), :])

    return run(idx, table, feat)


def kernel(edge_type, edge_feat, table):
    idx = edge_type.astype(jnp.int32)
    return _sc_embed_concat(idx, table, edge_feat)


# speedup vs baseline: 1.0225x; 1.0225x over previous
import functools

import jax
import jax.numpy as jnp
from jax import lax
from jax.experimental import pallas as pl
from jax.experimental.pallas import tpu as pltpu
from jax.experimental.pallas import tpu_sc as plsc

E = 320000
D_EMB = 128
D_FEAT = 16
D_OUT = D_EMB + D_FEAT
V = 400
BLK = 128
NBLK = E // BLK
NW = 32
NJ = NBLK // NW
L = 16


def _sc_embed_concat(idx, table, feat2):
    mesh = plsc.VectorSubcoreMesh(core_axis_name="core", subcore_axis_name="subcore")

    @functools.partial(
        pl.kernel,
        out_type=jax.ShapeDtypeStruct((E, D_OUT), jnp.float32),
        mesh=mesh,
        scratch_types=[
            pltpu.VMEM((V, D_EMB), jnp.float32),
            pltpu.VMEM((2, BLK), jnp.int32),
            pltpu.VMEM((2, BLK, D_OUT), jnp.float32),
            pltpu.VMEM((2, BLK // 8, 128), jnp.float32),
        ]
        + [pltpu.SemaphoreType.DMA] * 6,
    )
    def run(i_hbm, t_hbm, f_hbm, o_hbm, t_v, i_v, o_v, f_v,
            is0, is1, fs0, fs1, ws0, ws1):
        wid = lax.axis_index("subcore") * 2 + lax.axis_index("core")
        isems, fsems, wsems = (is0, is1), (fs0, fs1), (ws0, ws1)

        pltpu.sync_copy(t_hbm, t_v)

        def blk_of(j):
            return wid + NW * j

        def start_loads(j, p):
            b = blk_of(j)
            pltpu.async_copy(i_hbm.at[pl.ds(b * BLK, BLK)], i_v.at[p], isems[p])
            pltpu.async_copy(
                f_hbm.at[pl.ds(b * (BLK // 8), BLK // 8), :], f_v.at[p], fsems[p]
            )

        def wait_loads(j, p):
            b = blk_of(j)
            pltpu.make_async_copy(
                i_hbm.at[pl.ds(b * BLK, BLK)], i_v.at[p], isems[p]
            ).wait()
            pltpu.make_async_copy(
                f_hbm.at[pl.ds(b * (BLK // 8), BLK // 8), :], f_v.at[p], fsems[p]
            ).wait()

        def assemble(p):
            @pl.loop(0, BLK // L)
            def _(g):
                iv = i_v[p, pl.ds(g * L, L)]

                def stores(st):
                    vals, fval, r = st
                    for k in range(D_EMB // L):
                        o_v[p, r, pl.ds(k * L, L)] = vals[k]
                    o_v[p, r, pl.ds(D_EMB, D_FEAT)] = fval

                prev = None
                for e in range(L):
                    r = g * L + e
                    row = iv[e]
                    vals = [t_v[row, pl.ds(k * L, L)] for k in range(D_EMB // L)]
                    fval = f_v[p, r // 8, pl.ds((r % 8) * D_FEAT, D_FEAT)]
                    if prev is not None:
                        stores(prev)
                    prev = (vals, fval, r)
                stores(prev)

        start_loads(0, 0)
        start_loads(1, 1)

        @pl.loop(0, NJ)
        def _(j):
            for p in range(2):

                @pl.when(j % 2 == p)
                def _():
                    @pl.when(j >= 2)
                    def _():
                        pltpu.make_async_copy(
                            o_v.at[p], o_hbm.at[pl.ds(0, BLK), :], wsems[p]
                        ).wait()

                    wait_loads(j, p)
                    assemble(p)
                    pltpu.async_copy(
                        o_v.at[p],
                        o_hbm.at[pl.ds(blk_of(j) * BLK, BLK), :],
                        wsems[p],
                    )

                    @pl.when(j + 2 < NJ)
                    def _():
                        start_loads(j + 2, p)

        for p in range(2):
            pltpu.make_async_copy(
                o_v.at[p], o_hbm.at[pl.ds(0, BLK), :], wsems[p]
            ).wait()

        @pl.when(wid < NBLK - NW * NJ)
        def _():
            b = NW * NJ + wid
            pltpu.sync_copy(i_hbm.at[pl.ds(b * BLK, BLK)], i_v.at[0])
            pltpu.sync_copy(
                f_hbm.at[pl.ds(b * (BLK // 8), BLK // 8), :], f_v.at[0]
            )
            assemble(0)
            pltpu.sync_copy(o_v.at[0], o_hbm.at[pl.ds(b * BLK, BLK), :])

    return run(idx, table, feat2)


def kernel(edge_type, edge_feat, table):
    idx = edge_type.astype(jnp.int32)
    feat2 = edge_feat.reshape(E // 8, 128)
    return _sc_embed_concat(idx, table, feat2)
